# Initial kernel scaffold; baseline (speedup 1.0000x reference)
import jax
import jax.numpy as jnp
from jax.experimental import pallas as pl


def kernel(x, indices, W1, b1, W2, b2):
    raise NotImplementedError("write your pallas kernel here")


# speedup vs baseline: 4.3662x; 4.3662x over previous
import functools

import jax
import jax.numpy as jnp
from jax import lax
from jax.experimental import pallas as pl
from jax.experimental.pallas import tpu as pltpu
from jax.experimental.pallas import tpu_sc as plsc

V = 100000
K = 16
IN_C = 128
OUT_C = 128
PARTIAL_C = 32
VK = V * K

NC = 2
NS = 16
NW = NC * NS
PER_W = VK // NW
CHUNK = 80
NCHUNK = PER_W // CHUNK


def _sc_gather(idx3, table):
    mesh = plsc.VectorSubcoreMesh(core_axis_name="c", subcore_axis_name="s")

    @functools.partial(
        pl.kernel,
        out_type=jax.ShapeDtypeStruct((VK, PARTIAL_C), jnp.float32),
        mesh=mesh,
        compiler_params=pltpu.CompilerParams(use_tc_tiling_on_sc=False),
        scratch_types=[
            pltpu.VMEM((NCHUNK, CHUNK), jnp.int32),
            pltpu.VMEM((CHUNK, PARTIAL_C), jnp.float32),
            pltpu.SemaphoreType.DMA,
        ],
    )
    def gather_k(idx_hbm, table_hbm, out_hbm, idx_v, buf_v, sem):
        wid = lax.axis_index("s") * NC + lax.axis_index("c")
        base = wid * PER_W
        pltpu.sync_copy(idx_hbm.at[wid], idx_v)

        def body(j, carry):
            pltpu.async_copy(table_hbm.at[idx_v.at[j]], buf_v, sem).wait()
            pltpu.sync_copy(buf_v, out_hbm.at[pl.ds(base + j * CHUNK, CHUNK)])
            return carry

        lax.fori_loop(0, NCHUNK, body, 0)

    return gather_k(idx3, table)


def _mlp_body(xf_ref, x_ref, W1_ref, b1_ref, W2_ref, b2_ref, out_ref):
    cdims = (((1,), (1,)), ((), ()))
    p = lax.dot_general(xf_ref[...], W1_ref[...], cdims,
                        preferred_element_type=jnp.float32) + b1_ref[...]
    W2 = W2_ref[...]
    out = lax.dot_general(p, W2[:, :PARTIAL_C], cdims,
                          preferred_element_type=jnp.float32)
    out += lax.dot_general(x_ref[:, PARTIAL_C:], W2[:, PARTIAL_C:], cdims,
                           preferred_element_type=jnp.float32)
    out_ref[...] = out + b2_ref[...]


def _tc_mlp(xf, x, W1, b1, W2, b2):
    R = 2000
    grid = (V // R,)
    return pl.pallas_call(
        _mlp_body,
        grid=grid,
        in_specs=[
            pl.BlockSpec((R, K * PARTIAL_C), lambda i: (i, 0)),
            pl.BlockSpec((R, IN_C), lambda i: (i, 0)),
            pl.BlockSpec((PARTIAL_C, K * PARTIAL_C), lambda i: (0, 0)),
            pl.BlockSpec((1, PARTIAL_C), lambda i: (0, 0)),
            pl.BlockSpec((OUT_C, IN_C), lambda i: (0, 0)),
            pl.BlockSpec((1, OUT_C), lambda i: (0, 0)),
        ],
        out_specs=pl.BlockSpec((R, OUT_C), lambda i: (i, 0)),
        out_shape=jax.ShapeDtypeStruct((V, OUT_C), jnp.float32),
    )(xf, x, W1, b1, W2, b2)


def kernel(x, indices, W1, b1, W2, b2):
    table = x[:, :PARTIAL_C]
    idx3 = indices.astype(jnp.int32).reshape(NW, NCHUNK, CHUNK)
    xf = _sc_gather(idx3, table)
    xf = xf.reshape(V, K * PARTIAL_C)
    return _tc_mlp(xf, x, W1, b1.reshape(1, PARTIAL_C), W2,
                   b2.reshape(1, OUT_C))
